# baseline (speedup 1.0000x reference)
import functools

import jax
import jax.numpy as jnp
from jax.experimental import pallas as pl
from jax.experimental.pallas import tpu as pltpu
from jax.experimental.pallas import tpu_sc as plsc

N_CORES = 2
N_SUB = 16
N_TILES = N_CORES * N_SUB
CHUNK = 128
NCH = 80
IDXD = 4
PADROWS = 8
ROWCHUNK = 80

_MESH = plsc.VectorSubcoreMesh(
    core_axis_name="c", subcore_axis_name="s", num_cores=N_CORES, num_subcores=N_SUB
)


def _deg_body(n_rowchunks, col2d, ones_hbm, zeros_hbm, out_hbm, idx_all,
              ones_v, acc_sh, semp, semw):
    cid = jax.lax.axis_index("c")
    sid = jax.lax.axis_index("s")
    chunk_base = (cid * N_SUB + sid) * NCH

    cp = pltpu.async_copy(col2d.at[pl.ds(chunk_base, NCH)], idx_all, semp)
    cpo = pltpu.async_copy(ones_hbm, ones_v, semw)

    @pl.loop(sid, n_rowchunks, step=N_SUB)
    def _(j):
        pltpu.sync_copy(zeros_hbm.at[pl.ds(j * ROWCHUNK, ROWCHUNK)],
                        acc_sh.at[pl.ds(j * ROWCHUNK, ROWCHUNK)])

    cp.wait()
    cpo.wait()
    plsc.subcore_barrier()

    window = 8

    @pl.loop(0, NCH)
    def _(j):
        pltpu.async_copy(ones_v, acc_sh.at[idx_all.at[j]], semw, add=True)

        @pl.when(j >= window)
        def _():
            pltpu.make_async_copy(ones_v, acc_sh.at[idx_all.at[j]], semw).wait()

    for _ in range(window):
        pltpu.make_async_copy(ones_v, acc_sh.at[idx_all.at[0]], semw).wait()

    plsc.subcore_barrier()

    @pl.loop(sid, n_rowchunks, step=N_SUB)
    def _(j):
        pltpu.sync_copy(acc_sh.at[pl.ds(j * ROWCHUNK, ROWCHUNK)],
                        out_hbm.at[cid, pl.ds(j * ROWCHUNK, ROWCHUNK)])


def _sc_degree(col2d, ones128, zeros16, n):
    n_rowchunks = n // ROWCHUNK
    body = functools.partial(_deg_body, n_rowchunks)
    return pl.kernel(
        body,
        out_type=jax.ShapeDtypeStruct((N_CORES, n, N_SUB), jnp.float32),
        mesh=_MESH,
        scratch_types=[
            pltpu.VMEM((NCH, CHUNK), jnp.int32),
            pltpu.VMEM((CHUNK, N_SUB), jnp.float32),
            pltpu.VMEM_SHARED((n + PADROWS, N_SUB), jnp.float32),
            pltpu.SemaphoreType.DMA,
            pltpu.SemaphoreType.DMA,
        ],
    )(col2d, ones128, zeros16)


def _hop_body(n_rowchunks, d, g_hbm, row1d, col2d, zeros_hbm, out_hbm,
              idx_c_all, ri0, ri1, ri2, ri3, rw0, rw1, acc_sh,
              semp, si0, si1, si2, si3, sg0, sg1, ss0, ss1):
    cid = jax.lax.axis_index("c")
    sid = jax.lax.axis_index("s")
    rowidx = (ri0, ri1, ri2, ri3)
    semi = (si0, si1, si2, si3)
    rows = (rw0, rw1)
    semg = (sg0, sg1)
    sems = (ss0, ss1)
    chunk_base = (cid * N_SUB + sid) * NCH
    edge_base = chunk_base * CHUNK

    cp_c = pltpu.async_copy(col2d.at[pl.ds(chunk_base, NCH)], idx_c_all, semp)
    for q in range(IDXD):
        pltpu.async_copy(row1d.at[pl.ds(edge_base + q * CHUNK, CHUNK)],
                         rowidx[q], semi[q])

    @pl.loop(sid, n_rowchunks, step=N_SUB)
    def _(j):
        pltpu.sync_copy(zeros_hbm.at[pl.ds(j * ROWCHUNK, ROWCHUNK)],
                        acc_sh.at[pl.ds(j * ROWCHUNK, ROWCHUNK)])

    cp_c.wait()
    plsc.subcore_barrier()

    def stage_a(j, b, q, first=False):
        if not first:
            pltpu.make_async_copy(rows[b], acc_sh.at[idx_c_all.at[j - 2]],
                                  sems[b]).wait()
        pltpu.make_async_copy(row1d.at[pl.ds(edge_base + j * CHUNK, CHUNK)],
                              rowidx[q], semi[q]).wait()
        pltpu.async_copy(g_hbm.at[rowidx[q]], rows[b], semg[b])

    def stage_b(j, b, q):
        pltpu.make_async_copy(g_hbm.at[rowidx[q]], rows[b], semg[b]).wait()
        pltpu.async_copy(rows[b], acc_sh.at[idx_c_all.at[j]], sems[b],
                         add=True)

        @pl.when(j < NCH - IDXD)
        def _():
            pltpu.async_copy(
                row1d.at[pl.ds(edge_base + (j + IDXD) * CHUNK, CHUNK)],
                rowidx[q], semi[q])

    stage_a(0, 0, 0, first=True)
    stage_a(1, 1, 1, first=True)
    stage_b(0, 0, 0)
    stage_a(2, 0, 2)
    stage_b(1, 1, 1)
    stage_a(3, 1, 3)
    stage_b(2, 0, 2)

    @pl.loop(IDXD, NCH, step=IDXD)
    def _(j0):
        for t in range(IDXD):
            stage_a(j0 + t, t % 2, t)
            stage_b(j0 + t - 1, (t - 1) % 2, (t - 1) % IDXD)

    stage_b(NCH - 1, (NCH - 1) % 2, (NCH - 1) % IDXD)
    for j in (NCH - 2, NCH - 1):
        b = j % 2
        pltpu.make_async_copy(rows[b], acc_sh.at[idx_c_all.at[j]],
                              sems[b]).wait()

    plsc.subcore_barrier()

    @pl.loop(sid, n_rowchunks, step=N_SUB)
    def _(j):
        pltpu.sync_copy(acc_sh.at[pl.ds(j * ROWCHUNK, ROWCHUNK)],
                        out_hbm.at[cid, pl.ds(j * ROWCHUNK, ROWCHUNK)])


def _sc_hop(g, row1d, col2d, zeros_nd, n, d):
    n_rowchunks = n // ROWCHUNK
    body = functools.partial(_hop_body, n_rowchunks, d)
    return pl.kernel(
        body,
        out_type=jax.ShapeDtypeStruct((N_CORES, n, d), jnp.float32),
        mesh=_MESH,
        scratch_types=[
            pltpu.VMEM((NCH, CHUNK), jnp.int32),
            pltpu.VMEM((CHUNK,), jnp.int32),
            pltpu.VMEM((CHUNK,), jnp.int32),
            pltpu.VMEM((CHUNK,), jnp.int32),
            pltpu.VMEM((CHUNK,), jnp.int32),
            pltpu.VMEM((CHUNK, d), jnp.float32),
            pltpu.VMEM((CHUNK, d), jnp.float32),
            pltpu.VMEM_SHARED((n + PADROWS, d), jnp.float32),
            pltpu.SemaphoreType.DMA,
            pltpu.SemaphoreType.DMA,
            pltpu.SemaphoreType.DMA,
            pltpu.SemaphoreType.DMA,
            pltpu.SemaphoreType.DMA,
            pltpu.SemaphoreType.DMA,
            pltpu.SemaphoreType.DMA,
            pltpu.SemaphoreType.DMA,
            pltpu.SemaphoreType.DMA,
        ],
    )(g, row1d, col2d, zeros_nd)


_BLK = 1000


def _mm_body(x_ref, w_ref, y_ref):
    y_ref[...] = jax.lax.dot_general(
        x_ref[...], w_ref[...], (((1,), (1,)), ((), ())),
        preferred_element_type=jnp.float32)


def _tc_matmul(x, w, n, d):
    grid = n // _BLK
    return pl.pallas_call(
        _mm_body,
        grid=(grid,),
        in_specs=[
            pl.BlockSpec((_BLK, d), lambda i: (i, 0)),
            pl.BlockSpec((d, d), lambda i: (0, 0)),
        ],
        out_specs=pl.BlockSpec((_BLK, d), lambda i: (i, 0)),
        out_shape=jax.ShapeDtypeStruct((n, d), jnp.float32),
    )(x, w)


def _scale_body(degp_ref, y_ref, g0_ref, dinv_ref, invdeg_ref):
    deg = degp_ref[0][:, 0:1] + degp_ref[1][:, 0:1] + 1.0
    dinv = jax.lax.rsqrt(deg)
    g0_ref[...] = y_ref[...] * dinv
    dinv_ref[...] = dinv
    invdeg_ref[...] = 1.0 / deg


def _tc_scale(degpart, y, n, d):
    grid = n // _BLK
    return pl.pallas_call(
        _scale_body,
        grid=(grid,),
        in_specs=[
            pl.BlockSpec((N_CORES, _BLK, N_SUB), lambda i: (0, i, 0)),
            pl.BlockSpec((_BLK, d), lambda i: (i, 0)),
        ],
        out_specs=[
            pl.BlockSpec((_BLK, d), lambda i: (i, 0)),
            pl.BlockSpec((_BLK, 1), lambda i: (i, 0)),
            pl.BlockSpec((_BLK, 1), lambda i: (i, 0)),
        ],
        out_shape=[
            jax.ShapeDtypeStruct((n, d), jnp.float32),
            jax.ShapeDtypeStruct((n, 1), jnp.float32),
            jax.ShapeDtypeStruct((n, 1), jnp.float32),
        ],
    )(degpart, y)


def _combine_body(part_ref, g_ref, scale_ref, out_ref):
    out_ref[...] = (part_ref[0] + part_ref[1] + g_ref[...]) * scale_ref[...]


def _tc_combine(part, g, scale, n, d):
    grid = n // _BLK
    return pl.pallas_call(
        _combine_body,
        grid=(grid,),
        in_specs=[
            pl.BlockSpec((N_CORES, _BLK, d), lambda i: (0, i, 0)),
            pl.BlockSpec((_BLK, d), lambda i: (i, 0)),
            pl.BlockSpec((_BLK, 1), lambda i: (i, 0)),
        ],
        out_specs=pl.BlockSpec((_BLK, d), lambda i: (i, 0)),
        out_shape=jax.ShapeDtypeStruct((n, d), jnp.float32),
    )(part, g, scale)


def _final_body(part_ref, g_ref, dinv_ref, b_ref, out_ref):
    h = (part_ref[0] + part_ref[1] + g_ref[...]) * dinv_ref[...]
    out_ref[...] = jnp.maximum(h + b_ref[...], 0.0)


def _tc_final(part, g, dinv, b2, n, d):
    grid = n // _BLK
    return pl.pallas_call(
        _final_body,
        grid=(grid,),
        in_specs=[
            pl.BlockSpec((N_CORES, _BLK, d), lambda i: (0, i, 0)),
            pl.BlockSpec((_BLK, d), lambda i: (i, 0)),
            pl.BlockSpec((_BLK, 1), lambda i: (i, 0)),
            pl.BlockSpec((1, d), lambda i: (0, 0)),
        ],
        out_specs=pl.BlockSpec((_BLK, d), lambda i: (i, 0)),
        out_shape=jax.ShapeDtypeStruct((n, d), jnp.float32),
    )(part, g, dinv, b2)


def kernel(x, edge_index, W, b):
    n, d = x.shape
    e = edge_index.shape[1]
    e_tile = e // N_TILES
    seg = NCH * CHUNK
    pad = seg - e_tile
    assert e % N_TILES == 0 and 0 <= pad < seg
    assert n % ROWCHUNK == 0 and n % _BLK == 0

    idt = edge_index.dtype
    prow = jnp.zeros((N_TILES, pad), idt)
    pcol = n + (jnp.arange(pad, dtype=idt) % PADROWS)
    pcol = jnp.broadcast_to(pcol, (N_TILES, pad))
    row1d = jnp.concatenate(
        [edge_index[0].reshape(N_TILES, e_tile), prow], axis=1).reshape(-1)
    col2d = jnp.concatenate(
        [edge_index[1].reshape(N_TILES, e_tile), pcol], axis=1
    ).reshape(-1, CHUNK)

    zeros16 = jnp.zeros((n, N_SUB), jnp.float32)
    zeros_nd = jnp.zeros((n, d), jnp.float32)
    ones128 = jnp.ones((CHUNK, N_SUB), jnp.float32)

    degpart = _sc_degree(col2d, ones128, zeros16, n)
    y = _tc_matmul(x, W, n, d)
    g0, dinv, invdeg = _tc_scale(degpart, y, n, d)
    part1 = _sc_hop(g0, row1d, col2d, zeros_nd, n, d)
    g1 = _tc_combine(part1, g0, invdeg, n, d)
    part2 = _sc_hop(g1, row1d, col2d, zeros_nd, n, d)
    return _tc_final(part2, g1, dinv, b.reshape(1, d), n, d)
